# Initial kernel scaffold; baseline (speedup 1.0000x reference)
import jax
import jax.numpy as jnp
from jax.experimental import pallas as pl


def kernel(p1_x, p1_edge_index, p1_edge_attr, p1_num_routes, p1_client_route_vector, p2_x, p2_edge_index, p2_edge_attr, p2_num_routes, p2_client_route_vector, gat_W, att_src, att_dst, att_edge, lin_edge_W, gat_bias, head_W, head_b):
    raise NotImplementedError("write your pallas kernel here")


# speedup vs baseline: 4.3938x; 4.3938x over previous
import functools

import jax
import jax.numpy as jnp
from jax.experimental import pallas as pl
from jax.experimental.pallas import tpu as pltpu

H = 8
C = 64
NEG_SLOPE = 0.2
NUM_R1 = 50
NUM_R2 = 55
RPAD = 64
BE = 1000
NB = 2000


def _k1_body(x_ref, w_ref, wsrc_ref, wdst_ref, xl_ref, asrc_ref, adst_ref):
    x = x_ref[...]
    xl_ref[...] = jnp.dot(x, w_ref[...], preferred_element_type=jnp.float32)
    asrc_ref[...] = jnp.dot(x, wsrc_ref[...], preferred_element_type=jnp.float32)
    adst_ref[...] = jnp.dot(x, wdst_ref[...], preferred_element_type=jnp.float32)


def _k2_body(src_ref, dst_ref, asrc_ref, adst_ref, ea_ref, wedge_ref,
             ex_ref, den_ref, bufs_ref, bufd_ref):
    @pl.when(pl.program_id(0) == 0)
    def _():
        den_ref[...] = jnp.zeros_like(den_ref)

    a_edge = jnp.dot(ea_ref[...], wedge_ref[...],
                     preferred_element_type=jnp.float32)

    def gat(e, carry):
        s = src_ref[0, 0, e]
        d = dst_ref[0, 0, e]
        bufs_ref[pl.ds(e, 1), :] = asrc_ref[pl.ds(s, 1), :]
        bufd_ref[pl.ds(e, 1), :] = adst_ref[pl.ds(d, 1), :]
        return carry

    jax.lax.fori_loop(0, BE, gat, 0)
    alpha = bufs_ref[...] + bufd_ref[...] + a_edge
    alpha = jnp.where(alpha >= 0, alpha, NEG_SLOPE * alpha)
    ex = jnp.exp(alpha)
    ex_ref[...] = ex

    def scat(e, carry):
        d = dst_ref[0, 0, e]
        den_ref[pl.ds(d, 1), :] = den_ref[pl.ds(d, 1), :] + ex_ref[pl.ds(e, 1), :]
        return carry

    jax.lax.fori_loop(0, BE, scat, 0)


def _k3_body(src_ref, dst_ref, xl_ref, den_ref, ex_ref, bcast_ref,
             emb_ref, msg_ref, dbuf_ref):
    @pl.when(pl.program_id(0) == 0)
    def _():
        emb_ref[...] = jnp.zeros_like(emb_ref)

    def gat(e, carry):
        s = src_ref[0, 0, e]
        d = dst_ref[0, 0, e]
        msg_ref[pl.ds(e, 1), :] = xl_ref[pl.ds(s, 1), :]
        dbuf_ref[pl.ds(e, 1), :] = den_ref[pl.ds(d, 1), :]
        return carry

    jax.lax.fori_loop(0, BE, gat, 0)
    attn = ex_ref[...] / (dbuf_ref[...] + 1e-16)
    scale = jnp.dot(attn, bcast_ref[...], preferred_element_type=jnp.float32)
    msg_ref[...] = msg_ref[...] * scale

    def scat(e, carry):
        d = dst_ref[0, 0, e]
        emb_ref[pl.ds(d, 1), :] = emb_ref[pl.ds(d, 1), :] + msg_ref[pl.ds(e, 1), :]
        return carry

    jax.lax.fori_loop(0, BE, scat, 0)


def _k4_body(emb_ref, rv_ref, bias_ref, out_ref):
    @pl.when(pl.program_id(0) == 0)
    def _():
        out_ref[...] = jnp.zeros_like(out_ref)

    nb = emb_ref.shape[0]
    iota = jax.lax.broadcasted_iota(jnp.int32, (nb, RPAD), 1)
    oh = jnp.where(rv_ref[...] == iota, 1.0, 0.0).astype(jnp.float32)
    x = emb_ref[...] + bias_ref[...]
    out_ref[...] += jax.lax.dot_general(
        oh, x, (((0,), (0,)), ((), ())), preferred_element_type=jnp.float32)


def _k5_body(p1r_ref, p2r_ref, hw1_ref, hw2_ref, hb_ref, out_ref):
    a = jnp.dot(p1r_ref[...], hw1_ref[...],
                preferred_element_type=jnp.float32) + hb_ref[...]
    b = jnp.dot(p2r_ref[...], hw2_ref[...],
                preferred_element_type=jnp.float32)
    full = a[:, None, :] + b[None, :, :]
    ii = jax.lax.broadcasted_iota(jnp.int32, full.shape, 0)
    jj = jax.lax.broadcasted_iota(jnp.int32, full.shape, 1)
    kk = jax.lax.broadcasted_iota(jnp.int32, full.shape, 2)
    mask = (ii < NUM_R1) & (jj < NUM_R2) & (kk < NUM_R1 - 1)
    val = jnp.where(mask, full, -1e30)
    m = jnp.max(val)
    p = jnp.where(mask, jnp.exp(val - m), 0.0)
    out_ref[...] = p / jnp.sum(p)


def _gat_embed(x, edge_index, edge_attr, W, wsrc, wdst, wedge):
    N, F = x.shape
    E = edge_index.shape[1]
    HC = H * C
    nbc = N // NB if N % NB == 0 else 1
    nbs = N // nbc
    xl, asrc, adst = pl.pallas_call(
        _k1_body,
        grid=(nbc,),
        in_specs=[
            pl.BlockSpec((nbs, F), lambda i: (i, 0)),
            pl.BlockSpec((F, HC), lambda i: (0, 0)),
            pl.BlockSpec((F, H), lambda i: (0, 0)),
            pl.BlockSpec((F, H), lambda i: (0, 0)),
        ],
        out_specs=[
            pl.BlockSpec((nbs, HC), lambda i: (i, 0)),
            pl.BlockSpec((nbs, H), lambda i: (i, 0)),
            pl.BlockSpec((nbs, H), lambda i: (i, 0)),
        ],
        out_shape=[
            jax.ShapeDtypeStruct((N, HC), jnp.float32),
            jax.ShapeDtypeStruct((N, H), jnp.float32),
            jax.ShapeDtypeStruct((N, H), jnp.float32),
        ],
    )(x, W, wsrc, wdst)

    nec = E // BE
    src = edge_index[0].reshape(nec, 1, BE)
    dst = edge_index[1].reshape(nec, 1, BE)
    smem_spec = pl.BlockSpec((1, 1, BE), lambda i: (i, 0, 0),
                             memory_space=pltpu.SMEM)
    ex, den = pl.pallas_call(
        _k2_body,
        grid=(nec,),
        in_specs=[
            smem_spec,
            smem_spec,
            pl.BlockSpec((N, H), lambda i: (0, 0)),
            pl.BlockSpec((N, H), lambda i: (0, 0)),
            pl.BlockSpec((BE, edge_attr.shape[1]), lambda i: (i, 0)),
            pl.BlockSpec((edge_attr.shape[1], H), lambda i: (0, 0)),
        ],
        out_specs=[
            pl.BlockSpec((BE, H), lambda i: (i, 0)),
            pl.BlockSpec((N, H), lambda i: (0, 0)),
        ],
        out_shape=[
            jax.ShapeDtypeStruct((E, H), jnp.float32),
            jax.ShapeDtypeStruct((N, H), jnp.float32),
        ],
        scratch_shapes=[
            pltpu.VMEM((BE, H), jnp.float32),
            pltpu.VMEM((BE, H), jnp.float32),
        ],
    )(src, dst, asrc, adst, edge_attr, wedge)

    bcast = jnp.kron(jnp.eye(H, dtype=jnp.float32),
                     jnp.ones((1, C), dtype=jnp.float32))
    emb = pl.pallas_call(
        _k3_body,
        grid=(nec,),
        in_specs=[
            smem_spec,
            smem_spec,
            pl.BlockSpec((N, HC), lambda i: (0, 0)),
            pl.BlockSpec((N, H), lambda i: (0, 0)),
            pl.BlockSpec((BE, H), lambda i: (i, 0)),
            pl.BlockSpec((H, HC), lambda i: (0, 0)),
        ],
        out_specs=pl.BlockSpec((N, HC), lambda i: (0, 0)),
        out_shape=jax.ShapeDtypeStruct((N, HC), jnp.float32),
        scratch_shapes=[
            pltpu.VMEM((BE, HC), jnp.float32),
            pltpu.VMEM((BE, H), jnp.float32),
        ],
    )(src, dst, xl, den, ex, bcast)
    return emb


def _route_agg(emb, route_vec, bias):
    N, HC = emb.shape
    nbc = N // NB if N % NB == 0 else 1
    nbs = N // nbc
    rv = route_vec.reshape(N, 1)
    return pl.pallas_call(
        _k4_body,
        grid=(nbc,),
        in_specs=[
            pl.BlockSpec((nbs, HC), lambda i: (i, 0)),
            pl.BlockSpec((nbs, 1), lambda i: (i, 0)),
            pl.BlockSpec((1, HC), lambda i: (0, 0)),
        ],
        out_specs=pl.BlockSpec((RPAD, HC), lambda i: (0, 0)),
        out_shape=jax.ShapeDtypeStruct((RPAD, HC), jnp.float32),
    )(emb, rv, bias.reshape(1, HC))


@jax.jit
def kernel(p1_x, p1_edge_index, p1_edge_attr, p1_num_routes,
           p1_client_route_vector, p2_x, p2_edge_index, p2_edge_attr,
           p2_num_routes, p2_client_route_vector, gat_W, att_src, att_dst,
           att_edge, lin_edge_W, gat_bias, head_W, head_b):
    F = p1_x.shape[1]
    DE = p1_edge_attr.shape[1]
    HC = H * C
    wsrc = (gat_W.reshape(F, H, C) * att_src[None]).sum(-1)
    wdst = (gat_W.reshape(F, H, C) * att_dst[None]).sum(-1)
    wedge = (lin_edge_W.reshape(DE, H, C) * att_edge[None]).sum(-1)

    p1_emb = _gat_embed(p1_x, p1_edge_index, p1_edge_attr, gat_W,
                        wsrc, wdst, wedge)
    p2_emb = _gat_embed(p2_x, p2_edge_index, p2_edge_attr, gat_W,
                        wsrc, wdst, wedge)

    p1r = _route_agg(p1_emb, p1_client_route_vector, gat_bias)
    p2r = _route_agg(p2_emb, p2_client_route_vector, gat_bias)

    nheads = head_W.shape[1]
    hwpad = jnp.zeros((2 * HC, RPAD), jnp.float32).at[:, :nheads].set(head_W)
    hbpad = jnp.zeros((1, RPAD), jnp.float32).at[0, :nheads].set(head_b)

    probs = pl.pallas_call(
        _k5_body,
        in_specs=[
            pl.BlockSpec((RPAD, HC), lambda: (0, 0)),
            pl.BlockSpec((RPAD, HC), lambda: (0, 0)),
            pl.BlockSpec((HC, RPAD), lambda: (0, 0)),
            pl.BlockSpec((HC, RPAD), lambda: (0, 0)),
            pl.BlockSpec((1, RPAD), lambda: (0, 0)),
        ],
        out_specs=pl.BlockSpec((RPAD, RPAD, RPAD), lambda: (0, 0, 0)),
        out_shape=jax.ShapeDtypeStruct((RPAD, RPAD, RPAD), jnp.float32),
    )(p1r, p2r, hwpad[:HC], hwpad[HC:], hbpad)
    return probs[:NUM_R1, :NUM_R2, :NUM_R1 - 1]


# speedup vs baseline: 9.3700x; 2.1326x over previous
import functools

import jax
import jax.numpy as jnp
from jax.experimental import pallas as pl
from jax.experimental.pallas import tpu as pltpu

H = 8
C = 64
NEG_SLOPE = 0.2
NUM_R1 = 50
NUM_R2 = 55
RPAD = 64
BE = 1000
NB = 2000


def _k1_body(x_ref, w_ref, wsrc_ref, wdst_ref, xl_ref, asrc_ref, adst_ref):
    x = x_ref[...]
    xl_ref[...] = jnp.dot(x, w_ref[...], preferred_element_type=jnp.float32)
    asrc_ref[...] = jnp.dot(x, wsrc_ref[...], preferred_element_type=jnp.float32)
    adst_ref[...] = jnp.dot(x, wdst_ref[...], preferred_element_type=jnp.float32)


def _k2_body(src_ref, dst_ref, asrc_ref, adst_ref, ea_ref, wedge_ref,
             ex_ref, den_ref, bufs_ref, bufd_ref):
    @pl.when(pl.program_id(0) == 0)
    def _():
        den_ref[...] = jnp.zeros_like(den_ref)

    a_edge = jnp.dot(ea_ref[...], wedge_ref[...],
                     preferred_element_type=jnp.float32)

    def gat(e, carry):
        s = src_ref[0, 0, e]
        d = dst_ref[0, 0, e]
        bufs_ref[pl.ds(e, 1), :] = asrc_ref[pl.ds(s, 1), :]
        bufd_ref[pl.ds(e, 1), :] = adst_ref[pl.ds(d, 1), :]
        return carry

    jax.lax.fori_loop(0, BE, gat, 0, unroll=8)
    alpha = bufs_ref[...] + bufd_ref[...] + a_edge
    alpha = jnp.where(alpha >= 0, alpha, NEG_SLOPE * alpha)
    ex = jnp.exp(alpha)
    ex_ref[...] = ex

    def scat(e, carry):
        d = dst_ref[0, 0, e]
        den_ref[pl.ds(d, 1), :] = den_ref[pl.ds(d, 1), :] + ex_ref[pl.ds(e, 1), :]
        return carry

    jax.lax.fori_loop(0, BE, scat, 0, unroll=8)


def _k3_body(src_ref, dst_ref, xl_ref, den_ref, ex_ref, bcast_ref,
             emb_ref, scale_ref, dbuf_ref):
    @pl.when(pl.program_id(0) == 0)
    def _():
        emb_ref[...] = jnp.zeros_like(emb_ref)

    def gatden(e, carry):
        d = dst_ref[0, 0, e]
        dbuf_ref[pl.ds(e, 1), :] = den_ref[pl.ds(d, 1), :]
        return carry

    jax.lax.fori_loop(0, BE, gatden, 0, unroll=8)
    attn = ex_ref[...] / (dbuf_ref[...] + 1e-16)
    scale_ref[...] = jnp.dot(attn, bcast_ref[...],
                             preferred_element_type=jnp.float32)

    def scat(e, carry):
        s = src_ref[0, 0, e]
        d = dst_ref[0, 0, e]
        emb_ref[pl.ds(d, 1), :] = (emb_ref[pl.ds(d, 1), :] +
                                   xl_ref[pl.ds(s, 1), :] *
                                   scale_ref[pl.ds(e, 1), :])
        return carry

    jax.lax.fori_loop(0, BE, scat, 0, unroll=4)


def _k4_body(emb_ref, rv_ref, bias_ref, out_ref):
    @pl.when(pl.program_id(0) == 0)
    def _():
        out_ref[...] = jnp.zeros_like(out_ref)

    nb = emb_ref.shape[0]
    iota = jax.lax.broadcasted_iota(jnp.int32, (nb, RPAD), 1)
    oh = jnp.where(rv_ref[...] == iota, 1.0, 0.0).astype(jnp.float32)
    x = emb_ref[...] + bias_ref[...]
    out_ref[...] += jax.lax.dot_general(
        oh, x, (((0,), (0,)), ((), ())), preferred_element_type=jnp.float32)


def _k5_body(p1r_ref, p2r_ref, hw1_ref, hw2_ref, hb_ref, out_ref):
    a = jnp.dot(p1r_ref[...], hw1_ref[...],
                preferred_element_type=jnp.float32) + hb_ref[...]
    b = jnp.dot(p2r_ref[...], hw2_ref[...],
                preferred_element_type=jnp.float32)
    full = a[:, None, :] + b[None, :, :]
    ii = jax.lax.broadcasted_iota(jnp.int32, full.shape, 0)
    jj = jax.lax.broadcasted_iota(jnp.int32, full.shape, 1)
    kk = jax.lax.broadcasted_iota(jnp.int32, full.shape, 2)
    mask = (ii < NUM_R1) & (jj < NUM_R2) & (kk < NUM_R1 - 1)
    val = jnp.where(mask, full, -1e30)
    m = jnp.max(val)
    p = jnp.where(mask, jnp.exp(val - m), 0.0)
    out_ref[...] = p / jnp.sum(p)


def _gat_embed(x, edge_index, edge_attr, W, wsrc, wdst, wedge):
    N, F = x.shape
    E = edge_index.shape[1]
    HC = H * C
    nbc = N // NB if N % NB == 0 else 1
    nbs = N // nbc
    xl, asrc, adst = pl.pallas_call(
        _k1_body,
        grid=(nbc,),
        in_specs=[
            pl.BlockSpec((nbs, F), lambda i: (i, 0)),
            pl.BlockSpec((F, HC), lambda i: (0, 0)),
            pl.BlockSpec((F, H), lambda i: (0, 0)),
            pl.BlockSpec((F, H), lambda i: (0, 0)),
        ],
        out_specs=[
            pl.BlockSpec((nbs, HC), lambda i: (i, 0)),
            pl.BlockSpec((nbs, H), lambda i: (i, 0)),
            pl.BlockSpec((nbs, H), lambda i: (i, 0)),
        ],
        out_shape=[
            jax.ShapeDtypeStruct((N, HC), jnp.float32),
            jax.ShapeDtypeStruct((N, H), jnp.float32),
            jax.ShapeDtypeStruct((N, H), jnp.float32),
        ],
    )(x, W, wsrc, wdst)

    nec = E // BE
    src = edge_index[0].reshape(nec, 1, BE)
    dst = edge_index[1].reshape(nec, 1, BE)
    smem_spec = pl.BlockSpec((1, 1, BE), lambda i: (i, 0, 0),
                             memory_space=pltpu.SMEM)
    ex, den = pl.pallas_call(
        _k2_body,
        grid=(nec,),
        in_specs=[
            smem_spec,
            smem_spec,
            pl.BlockSpec((N, H), lambda i: (0, 0)),
            pl.BlockSpec((N, H), lambda i: (0, 0)),
            pl.BlockSpec((BE, edge_attr.shape[1]), lambda i: (i, 0)),
            pl.BlockSpec((edge_attr.shape[1], H), lambda i: (0, 0)),
        ],
        out_specs=[
            pl.BlockSpec((BE, H), lambda i: (i, 0)),
            pl.BlockSpec((N, H), lambda i: (0, 0)),
        ],
        out_shape=[
            jax.ShapeDtypeStruct((E, H), jnp.float32),
            jax.ShapeDtypeStruct((N, H), jnp.float32),
        ],
        scratch_shapes=[
            pltpu.VMEM((BE, H), jnp.float32),
            pltpu.VMEM((BE, H), jnp.float32),
        ],
    )(src, dst, asrc, adst, edge_attr, wedge)

    bcast = jnp.kron(jnp.eye(H, dtype=jnp.float32),
                     jnp.ones((1, C), dtype=jnp.float32))
    emb = pl.pallas_call(
        _k3_body,
        grid=(nec,),
        in_specs=[
            smem_spec,
            smem_spec,
            pl.BlockSpec((N, HC), lambda i: (0, 0)),
            pl.BlockSpec((N, H), lambda i: (0, 0)),
            pl.BlockSpec((BE, H), lambda i: (i, 0)),
            pl.BlockSpec((H, HC), lambda i: (0, 0)),
        ],
        out_specs=pl.BlockSpec((N, HC), lambda i: (0, 0)),
        out_shape=jax.ShapeDtypeStruct((N, HC), jnp.float32),
        scratch_shapes=[
            pltpu.VMEM((BE, HC), jnp.float32),
            pltpu.VMEM((BE, H), jnp.float32),
        ],
    )(src, dst, xl, den, ex, bcast)
    return emb


def _route_agg(emb, route_vec, bias):
    N, HC = emb.shape
    nbc = N // NB if N % NB == 0 else 1
    nbs = N // nbc
    rv = route_vec.reshape(N, 1)
    return pl.pallas_call(
        _k4_body,
        grid=(nbc,),
        in_specs=[
            pl.BlockSpec((nbs, HC), lambda i: (i, 0)),
            pl.BlockSpec((nbs, 1), lambda i: (i, 0)),
            pl.BlockSpec((1, HC), lambda i: (0, 0)),
        ],
        out_specs=pl.BlockSpec((RPAD, HC), lambda i: (0, 0)),
        out_shape=jax.ShapeDtypeStruct((RPAD, HC), jnp.float32),
    )(emb, rv, bias.reshape(1, HC))


@jax.jit
def kernel(p1_x, p1_edge_index, p1_edge_attr, p1_num_routes,
           p1_client_route_vector, p2_x, p2_edge_index, p2_edge_attr,
           p2_num_routes, p2_client_route_vector, gat_W, att_src, att_dst,
           att_edge, lin_edge_W, gat_bias, head_W, head_b):
    F = p1_x.shape[1]
    DE = p1_edge_attr.shape[1]
    HC = H * C
    wsrc = (gat_W.reshape(F, H, C) * att_src[None]).sum(-1)
    wdst = (gat_W.reshape(F, H, C) * att_dst[None]).sum(-1)
    wedge = (lin_edge_W.reshape(DE, H, C) * att_edge[None]).sum(-1)

    p1_emb = _gat_embed(p1_x, p1_edge_index, p1_edge_attr, gat_W,
                        wsrc, wdst, wedge)
    p2_emb = _gat_embed(p2_x, p2_edge_index, p2_edge_attr, gat_W,
                        wsrc, wdst, wedge)

    p1r = _route_agg(p1_emb, p1_client_route_vector, gat_bias)
    p2r = _route_agg(p2_emb, p2_client_route_vector, gat_bias)

    nheads = head_W.shape[1]
    hwpad = jnp.zeros((2 * HC, RPAD), jnp.float32).at[:, :nheads].set(head_W)
    hbpad = jnp.zeros((1, RPAD), jnp.float32).at[0, :nheads].set(head_b)

    probs = pl.pallas_call(
        _k5_body,
        in_specs=[
            pl.BlockSpec((RPAD, HC), lambda: (0, 0)),
            pl.BlockSpec((RPAD, HC), lambda: (0, 0)),
            pl.BlockSpec((HC, RPAD), lambda: (0, 0)),
            pl.BlockSpec((HC, RPAD), lambda: (0, 0)),
            pl.BlockSpec((1, RPAD), lambda: (0, 0)),
        ],
        out_specs=pl.BlockSpec((RPAD, RPAD, RPAD), lambda: (0, 0, 0)),
        out_shape=jax.ShapeDtypeStruct((RPAD, RPAD, RPAD), jnp.float32),
    )(p1r, p2r, hwpad[:HC], hwpad[HC:], hbpad)
    return probs[:NUM_R1, :NUM_R2, :NUM_R1 - 1]


# speedup vs baseline: 10.2428x; 1.0931x over previous
import functools

import jax
import jax.numpy as jnp
from jax.experimental import pallas as pl
from jax.experimental.pallas import tpu as pltpu

H = 8
C = 64
NEG_SLOPE = 0.2
NUM_R1 = 50
NUM_R2 = 55
RPAD = 64
BE = 2000
NB = 2000


def _k1_body(x_ref, w_ref, wsrc_ref, wdst_ref, xl_ref, asrc_ref, adst_ref):
    x = x_ref[...]
    xl_ref[...] = jnp.dot(x, w_ref[...], preferred_element_type=jnp.float32)
    asrc_ref[...] = jnp.dot(x, wsrc_ref[...], preferred_element_type=jnp.float32)
    adst_ref[...] = jnp.dot(x, wdst_ref[...], preferred_element_type=jnp.float32)


def _k2_body(src_ref, dst_ref, asrc_ref, adst_ref, ea_ref, wedge_ref,
             ex_ref, den_ref, bufs_ref, bufd_ref):
    @pl.when(pl.program_id(0) == 0)
    def _():
        den_ref[...] = jnp.zeros_like(den_ref)

    a_edge = jnp.dot(ea_ref[...], wedge_ref[...],
                     preferred_element_type=jnp.float32)

    def gat(e, carry):
        s = src_ref[0, 0, e]
        d = dst_ref[0, 0, e]
        bufs_ref[pl.ds(e, 1), :] = asrc_ref[pl.ds(s, 1), :]
        bufd_ref[pl.ds(e, 1), :] = adst_ref[pl.ds(d, 1), :]
        return carry

    jax.lax.fori_loop(0, BE, gat, 0, unroll=8)
    alpha = bufs_ref[...] + bufd_ref[...] + a_edge
    alpha = jnp.where(alpha >= 0, alpha, NEG_SLOPE * alpha)
    ex = jnp.exp(alpha)
    ex_ref[...] = ex

    def scat(e, carry):
        d = dst_ref[0, 0, e]
        den_ref[pl.ds(d, 1), :] = den_ref[pl.ds(d, 1), :] + ex_ref[pl.ds(e, 1), :]
        return carry

    jax.lax.fori_loop(0, BE, scat, 0, unroll=8)


def _k3_body(src_ref, dst_ref, xl_ref, den_ref, ex_ref, bcast_ref,
             emb_ref, scale_ref, dbuf_ref):
    @pl.when(pl.program_id(0) == 0)
    def _():
        emb_ref[...] = jnp.zeros_like(emb_ref)

    def gatden(e, carry):
        d = dst_ref[0, 0, e]
        dbuf_ref[pl.ds(e, 1), :] = den_ref[pl.ds(d, 1), :]
        return carry

    jax.lax.fori_loop(0, BE, gatden, 0, unroll=8)
    attn = ex_ref[...] / (dbuf_ref[...] + 1e-16)
    scale_ref[...] = jnp.dot(attn, bcast_ref[...],
                             preferred_element_type=jnp.float32)

    def scat(e, carry):
        s = src_ref[0, 0, e]
        d = dst_ref[0, 0, e]
        emb_ref[pl.ds(d, 1), :] = (emb_ref[pl.ds(d, 1), :] +
                                   xl_ref[pl.ds(s, 1), :] *
                                   scale_ref[pl.ds(e, 1), :])
        return carry

    jax.lax.fori_loop(0, BE, scat, 0, unroll=8)


def _k4_body(emb_ref, rv_ref, bias_ref, out_ref):
    @pl.when(pl.program_id(0) == 0)
    def _():
        out_ref[...] = jnp.zeros_like(out_ref)

    nb = emb_ref.shape[0]
    iota = jax.lax.broadcasted_iota(jnp.int32, (nb, RPAD), 1)
    oh = jnp.where(rv_ref[...] == iota, 1.0, 0.0).astype(jnp.float32)
    x = emb_ref[...] + bias_ref[...]
    out_ref[...] += jax.lax.dot_general(
        oh, x, (((0,), (0,)), ((), ())), preferred_element_type=jnp.float32)


def _k5_body(p1r_ref, p2r_ref, hw1_ref, hw2_ref, hb_ref, out_ref):
    a = jnp.dot(p1r_ref[...], hw1_ref[...],
                preferred_element_type=jnp.float32) + hb_ref[...]
    b = jnp.dot(p2r_ref[...], hw2_ref[...],
                preferred_element_type=jnp.float32)
    full = a[:, None, :] + b[None, :, :]
    ii = jax.lax.broadcasted_iota(jnp.int32, full.shape, 0)
    jj = jax.lax.broadcasted_iota(jnp.int32, full.shape, 1)
    kk = jax.lax.broadcasted_iota(jnp.int32, full.shape, 2)
    mask = (ii < NUM_R1) & (jj < NUM_R2) & (kk < NUM_R1 - 1)
    val = jnp.where(mask, full, -1e30)
    m = jnp.max(val)
    p = jnp.where(mask, jnp.exp(val - m), 0.0)
    out_ref[...] = p / jnp.sum(p)


def _gat_embed(x, edge_index, edge_attr, W, wsrc, wdst, wedge):
    N, F = x.shape
    E = edge_index.shape[1]
    HC = H * C
    nbc = N // NB if N % NB == 0 else 1
    nbs = N // nbc
    xl, asrc, adst = pl.pallas_call(
        _k1_body,
        grid=(nbc,),
        in_specs=[
            pl.BlockSpec((nbs, F), lambda i: (i, 0)),
            pl.BlockSpec((F, HC), lambda i: (0, 0)),
            pl.BlockSpec((F, H), lambda i: (0, 0)),
            pl.BlockSpec((F, H), lambda i: (0, 0)),
        ],
        out_specs=[
            pl.BlockSpec((nbs, HC), lambda i: (i, 0)),
            pl.BlockSpec((nbs, H), lambda i: (i, 0)),
            pl.BlockSpec((nbs, H), lambda i: (i, 0)),
        ],
        out_shape=[
            jax.ShapeDtypeStruct((N, HC), jnp.float32),
            jax.ShapeDtypeStruct((N, H), jnp.float32),
            jax.ShapeDtypeStruct((N, H), jnp.float32),
        ],
    )(x, W, wsrc, wdst)

    nec = E // BE
    src = edge_index[0].reshape(nec, 1, BE)
    dst = edge_index[1].reshape(nec, 1, BE)
    smem_spec = pl.BlockSpec((1, 1, BE), lambda i: (i, 0, 0),
                             memory_space=pltpu.SMEM)
    ex, den = pl.pallas_call(
        _k2_body,
        grid=(nec,),
        in_specs=[
            smem_spec,
            smem_spec,
            pl.BlockSpec((N, H), lambda i: (0, 0)),
            pl.BlockSpec((N, H), lambda i: (0, 0)),
            pl.BlockSpec((BE, edge_attr.shape[1]), lambda i: (i, 0)),
            pl.BlockSpec((edge_attr.shape[1], H), lambda i: (0, 0)),
        ],
        out_specs=[
            pl.BlockSpec((BE, H), lambda i: (i, 0)),
            pl.BlockSpec((N, H), lambda i: (0, 0)),
        ],
        out_shape=[
            jax.ShapeDtypeStruct((E, H), jnp.float32),
            jax.ShapeDtypeStruct((N, H), jnp.float32),
        ],
        scratch_shapes=[
            pltpu.VMEM((BE, H), jnp.float32),
            pltpu.VMEM((BE, H), jnp.float32),
        ],
    )(src, dst, asrc, adst, edge_attr, wedge)

    bcast = jnp.kron(jnp.eye(H, dtype=jnp.float32),
                     jnp.ones((1, C), dtype=jnp.float32))
    emb = pl.pallas_call(
        _k3_body,
        grid=(nec,),
        in_specs=[
            smem_spec,
            smem_spec,
            pl.BlockSpec((N, HC), lambda i: (0, 0)),
            pl.BlockSpec((N, H), lambda i: (0, 0)),
            pl.BlockSpec((BE, H), lambda i: (i, 0)),
            pl.BlockSpec((H, HC), lambda i: (0, 0)),
        ],
        out_specs=pl.BlockSpec((N, HC), lambda i: (0, 0)),
        out_shape=jax.ShapeDtypeStruct((N, HC), jnp.float32),
        scratch_shapes=[
            pltpu.VMEM((BE, HC), jnp.float32),
            pltpu.VMEM((BE, H), jnp.float32),
        ],
    )(src, dst, xl, den, ex, bcast)
    return emb


def _route_agg(emb, route_vec, bias):
    N, HC = emb.shape
    nbc = N // NB if N % NB == 0 else 1
    nbs = N // nbc
    rv = route_vec.reshape(N, 1)
    return pl.pallas_call(
        _k4_body,
        grid=(nbc,),
        in_specs=[
            pl.BlockSpec((nbs, HC), lambda i: (i, 0)),
            pl.BlockSpec((nbs, 1), lambda i: (i, 0)),
            pl.BlockSpec((1, HC), lambda i: (0, 0)),
        ],
        out_specs=pl.BlockSpec((RPAD, HC), lambda i: (0, 0)),
        out_shape=jax.ShapeDtypeStruct((RPAD, HC), jnp.float32),
    )(emb, rv, bias.reshape(1, HC))


@jax.jit
def kernel(p1_x, p1_edge_index, p1_edge_attr, p1_num_routes,
           p1_client_route_vector, p2_x, p2_edge_index, p2_edge_attr,
           p2_num_routes, p2_client_route_vector, gat_W, att_src, att_dst,
           att_edge, lin_edge_W, gat_bias, head_W, head_b):
    F = p1_x.shape[1]
    DE = p1_edge_attr.shape[1]
    HC = H * C
    wsrc = (gat_W.reshape(F, H, C) * att_src[None]).sum(-1)
    wdst = (gat_W.reshape(F, H, C) * att_dst[None]).sum(-1)
    wedge = (lin_edge_W.reshape(DE, H, C) * att_edge[None]).sum(-1)

    p1_emb = _gat_embed(p1_x, p1_edge_index, p1_edge_attr, gat_W,
                        wsrc, wdst, wedge)
    p2_emb = _gat_embed(p2_x, p2_edge_index, p2_edge_attr, gat_W,
                        wsrc, wdst, wedge)

    p1r = _route_agg(p1_emb, p1_client_route_vector, gat_bias)
    p2r = _route_agg(p2_emb, p2_client_route_vector, gat_bias)

    nheads = head_W.shape[1]
    hwpad = jnp.zeros((2 * HC, RPAD), jnp.float32).at[:, :nheads].set(head_W)
    hbpad = jnp.zeros((1, RPAD), jnp.float32).at[0, :nheads].set(head_b)

    probs = pl.pallas_call(
        _k5_body,
        in_specs=[
            pl.BlockSpec((RPAD, HC), lambda: (0, 0)),
            pl.BlockSpec((RPAD, HC), lambda: (0, 0)),
            pl.BlockSpec((HC, RPAD), lambda: (0, 0)),
            pl.BlockSpec((HC, RPAD), lambda: (0, 0)),
            pl.BlockSpec((1, RPAD), lambda: (0, 0)),
        ],
        out_specs=pl.BlockSpec((RPAD, RPAD, RPAD), lambda: (0, 0, 0)),
        out_shape=jax.ShapeDtypeStruct((RPAD, RPAD, RPAD), jnp.float32),
    )(p1r, p2r, hwpad[:HC], hwpad[HC:], hbpad)
    return probs[:NUM_R1, :NUM_R2, :NUM_R1 - 1]
